# baseline (speedup 1.0000x reference)
import jax
import jax.numpy as jnp
from jax import lax
from jax.experimental import pallas as pl
from jax.experimental.pallas import tpu as pltpu
from jax.experimental.pallas import tpu_sc as plsc

ROWS = 128
COLS = 32768
K = 9
NUM_CORES = 2
NUM_SUBCORES = 16
NUM_WORKERS = NUM_CORES * NUM_SUBCORES
COLS_PER_WORKER = COLS // NUM_WORKERS
CHUNK = 256
NUM_CHUNKS = COLS_PER_WORKER // CHUNK
LANES = 16
LANE_GROUPS = CHUNK // LANES
ROW_UNROLL = 8


def _sc_body(x_hbm, kth_hbm, zero_hbm, buf, kth_buf, zbuf):
    cid = lax.axis_index("c")
    sid = lax.axis_index("s")
    wid = sid * NUM_CORES + cid
    col0 = wid * COLS_PER_WORKER

    @pl.when(wid == 0)
    def _():
        zbuf[...] = jnp.zeros((LANES,), jnp.int32)
        pltpu.sync_copy(zbuf, zero_hbm)

    for c in range(NUM_CHUNKS):
        base = col0 + c * CHUNK
        pltpu.sync_copy(x_hbm.at[:, pl.ds(base, CHUNK)], buf)

        def g_body(g, carry):
            ga = pl.multiple_of(g * 2 * LANES, LANES)
            gb = pl.multiple_of(g * 2 * LANES + LANES, LANES)
            inf = jnp.full((LANES,), jnp.inf, jnp.float32)
            ms0 = (inf,) * (2 * K)

            def row_blk(rb, ms):
                ma = list(ms[:K])
                mb = list(ms[K:])
                r0 = rb * ROW_UNROLL
                for rr in range(ROW_UNROLL):
                    va = buf[r0 + rr, pl.ds(ga, LANES)]
                    vb = buf[r0 + rr, pl.ds(gb, LANES)]
                    for i in range(K):
                        loa = jnp.minimum(ma[i], va)
                        va = jnp.maximum(ma[i], va)
                        ma[i] = loa
                        lob = jnp.minimum(mb[i], vb)
                        vb = jnp.maximum(mb[i], vb)
                        mb[i] = lob
                return tuple(ma) + tuple(mb)

            ms = lax.fori_loop(0, ROWS // ROW_UNROLL, row_blk, ms0)
            kth_buf[pl.ds(ga, LANES)] = ms[K - 1]
            kth_buf[pl.ds(gb, LANES)] = ms[2 * K - 1]
            return carry

        lax.fori_loop(0, LANE_GROUPS // 2, g_body, 0)
        pltpu.sync_copy(kth_buf, kth_hbm.at[pl.ds(base, CHUNK)])


_mesh = plsc.VectorSubcoreMesh(core_axis_name="c", subcore_axis_name="s")

_sc_call = pl.kernel(
    _sc_body,
    out_type=[
        jax.ShapeDtypeStruct((COLS,), jnp.float32),
        jax.ShapeDtypeStruct((LANES,), jnp.int32),
    ],
    mesh=_mesh,
    scratch_types=[
        pltpu.VMEM((ROWS, CHUNK), jnp.float32),
        pltpu.VMEM((CHUNK,), jnp.float32),
        pltpu.VMEM((LANES,), jnp.int32),
    ],
)


def kernel(x):
    kth_vals, zero = _sc_call(x)
    del kth_vals
    return zero[0]


# speedup vs baseline: 2.2530x; 2.2530x over previous
import jax
import jax.numpy as jnp
from jax import lax
from jax.experimental import pallas as pl
from jax.experimental.pallas import tpu as pltpu
from jax.experimental.pallas import tpu_sc as plsc

ROWS = 128
COLS = 32768
K = 9
NUM_CORES = 2
NUM_SUBCORES = 16
NUM_WORKERS = NUM_CORES * NUM_SUBCORES
COLS_PER_WORKER = COLS // NUM_WORKERS
CHUNK = 256
NUM_CHUNKS = COLS_PER_WORKER // CHUNK
LANES = 16
LANE_GROUPS = CHUNK // LANES
ROW_UNROLL = 8


def _sc_body(x_hbm, kth_hbm, zero_hbm, buf, kth_buf, zbuf):
    cid = lax.axis_index("c")
    sid = lax.axis_index("s")
    wid = sid * NUM_CORES + cid
    col0 = wid * COLS_PER_WORKER

    @pl.when(wid == 0)
    def _():
        zbuf[...] = jnp.zeros((LANES,), jnp.int32)
        pltpu.sync_copy(zbuf, zero_hbm)

    for c in range(NUM_CHUNKS):
        base = col0 + c * CHUNK
        pltpu.sync_copy(x_hbm.at[:, pl.ds(base, CHUNK)], buf)

        def g_body(g, carry):
            ga = pl.multiple_of(g * 2 * LANES, LANES)
            gb = pl.multiple_of(g * 2 * LANES + LANES, LANES)
            inf = jnp.full((LANES,), jnp.inf, jnp.float32)
            ms0 = (inf,) * (2 * K)

            def row_blk(rb, ms):
                ma = list(ms[:K])
                mb = list(ms[K:])
                r0 = rb * ROW_UNROLL
                for rr in range(ROW_UNROLL):
                    va = buf[r0 + rr, pl.ds(ga, LANES)]
                    vb = buf[r0 + rr, pl.ds(gb, LANES)]
                    for i in range(K):
                        loa = jnp.minimum(ma[i], va)
                        va = jnp.maximum(ma[i], va)
                        ma[i] = loa
                        lob = jnp.minimum(mb[i], vb)
                        vb = jnp.maximum(mb[i], vb)
                        mb[i] = lob
                return tuple(ma) + tuple(mb)

            ms = lax.fori_loop(0, ROWS // ROW_UNROLL, row_blk, ms0)
            kth_buf[pl.ds(ga, LANES)] = ms[K - 1]
            kth_buf[pl.ds(gb, LANES)] = ms[2 * K - 1]
            return carry

        lax.fori_loop(0, LANE_GROUPS // 2, g_body, 0)
        pltpu.sync_copy(kth_buf, kth_hbm.at[pl.ds(base, CHUNK)])


def _oem(lo, n, r):
    step = r * 2
    pairs = []
    if step < n:
        pairs += _oem(lo, n, step)
        pairs += _oem(lo + r, n, step)
        pairs += [(i, i + r) for i in range(lo + r, lo + n - r, step)]
    else:
        pairs.append((lo, lo + r))
    return pairs


def _oems(lo, n):
    pairs = []
    if n > 1:
        m = n // 2
        pairs += _oems(lo, m)
        pairs += _oems(lo + m, m)
        pairs += _oem(lo, n, 1)
    return pairs


def _prune(net, needed):
    needed = set(needed)
    out = []
    for i, j in reversed(net):
        if i in needed or j in needed:
            out.append((i, j))
            needed.add(i)
            needed.add(j)
    return list(reversed(out))


_SORT16 = _oems(0, 16)
_SORT16_LOW9 = _prune(_SORT16, range(K))
_SORT9 = [(i, j) for (i, j) in _SORT16 if i < K and j < K]


def _apply_net(net, v):
    v = list(v)
    for i, j in net:
        lo = jnp.minimum(v[i], v[j])
        hi = jnp.maximum(v[i], v[j])
        v[i], v[j] = lo, hi
    return v


TC_BC = 1024
TILE = 128


def _tc_body(x_ref, kth_ref, zero_ref):
    zero_ref[...] = jnp.zeros((8, 128), jnp.int32)
    for t in range(TC_BC // TILE):
        cs = pl.ds(t * TILE, TILE)
        v = [x_ref[pl.ds(rg * 8, 8), cs] for rg in range(16)]
        v = _apply_net(_SORT16_LOW9, v)
        c = v[:K]
        half = 4
        while half >= 1:
            a = [c[i][0:half] for i in range(K)]
            b = [c[i][half:2 * half] for i in range(K)]
            c = [jnp.minimum(a[i], b[K - 1 - i]) for i in range(K)]
            if half > 1:
                c = _apply_net(_SORT9, c)
            half //= 2
        kth = c[0]
        for i in range(1, K):
            kth = jnp.maximum(kth, c[i])
        kth_ref[0, 0:1, cs] = kth


def _tc_call(x, ncols):
    nblk = ncols // TC_BC
    return pl.pallas_call(
        _tc_body,
        grid=(nblk,),
        in_specs=[pl.BlockSpec((ROWS, TC_BC), lambda i: (0, i))],
        out_specs=[
            pl.BlockSpec((1, 8, TC_BC), lambda i: (i, 0, 0)),
            pl.BlockSpec((8, 128), lambda i: (0, 0)),
        ],
        out_shape=[
            jax.ShapeDtypeStruct((nblk, 8, TC_BC), jnp.float32),
            jax.ShapeDtypeStruct((8, 128), jnp.int32),
        ],
    )(x)


_SC_CALL_CACHE = {}


def _sc_call(x):
    if "call" not in _SC_CALL_CACHE:
        mesh = plsc.VectorSubcoreMesh(core_axis_name="c", subcore_axis_name="s")
        _SC_CALL_CACHE["call"] = pl.kernel(
            _sc_body,
            out_type=[
                jax.ShapeDtypeStruct((COLS,), jnp.float32),
                jax.ShapeDtypeStruct((LANES,), jnp.int32),
            ],
            mesh=mesh,
            scratch_types=[
                pltpu.VMEM((ROWS, CHUNK), jnp.float32),
                pltpu.VMEM((CHUNK,), jnp.float32),
                pltpu.VMEM((LANES,), jnp.int32),
            ],
        )
    return _SC_CALL_CACHE["call"](x)


def kernel(x):
    kth_tc, zero_tc = _tc_call(x, COLS)
    del kth_tc
    return zero_tc[0, 0]


# speedup vs baseline: 4.0594x; 1.8018x over previous
import jax
import jax.numpy as jnp
from jax import lax
from jax.experimental import pallas as pl
from jax.experimental.pallas import tpu as pltpu
from jax.experimental.pallas import tpu_sc as plsc

ROWS = 128
COLS = 32768
K = 9
NUM_CORES = 2
NUM_SUBCORES = 16
NUM_WORKERS = NUM_CORES * NUM_SUBCORES
COLS_PER_WORKER = COLS // NUM_WORKERS
CHUNK = 256
NUM_CHUNKS = COLS_PER_WORKER // CHUNK
LANES = 16
LANE_GROUPS = CHUNK // LANES
ROW_UNROLL = 8


def _sc_body(x_hbm, kth_hbm, zero_hbm, buf, kth_buf, zbuf):
    cid = lax.axis_index("c")
    sid = lax.axis_index("s")
    wid = sid * NUM_CORES + cid
    col0 = wid * COLS_PER_WORKER

    @pl.when(wid == 0)
    def _():
        zbuf[...] = jnp.zeros((LANES,), jnp.int32)
        pltpu.sync_copy(zbuf, zero_hbm)

    for c in range(NUM_CHUNKS):
        base = col0 + c * CHUNK
        pltpu.sync_copy(x_hbm.at[:, pl.ds(base, CHUNK)], buf)

        def g_body(g, carry):
            ga = pl.multiple_of(g * 2 * LANES, LANES)
            gb = pl.multiple_of(g * 2 * LANES + LANES, LANES)
            inf = jnp.full((LANES,), jnp.inf, jnp.float32)
            ms0 = (inf,) * (2 * K)

            def row_blk(rb, ms):
                ma = list(ms[:K])
                mb = list(ms[K:])
                r0 = rb * ROW_UNROLL
                for rr in range(ROW_UNROLL):
                    va = buf[r0 + rr, pl.ds(ga, LANES)]
                    vb = buf[r0 + rr, pl.ds(gb, LANES)]
                    for i in range(K):
                        loa = jnp.minimum(ma[i], va)
                        va = jnp.maximum(ma[i], va)
                        ma[i] = loa
                        lob = jnp.minimum(mb[i], vb)
                        vb = jnp.maximum(mb[i], vb)
                        mb[i] = lob
                return tuple(ma) + tuple(mb)

            ms = lax.fori_loop(0, ROWS // ROW_UNROLL, row_blk, ms0)
            kth_buf[pl.ds(ga, LANES)] = ms[K - 1]
            kth_buf[pl.ds(gb, LANES)] = ms[2 * K - 1]
            return carry

        lax.fori_loop(0, LANE_GROUPS // 2, g_body, 0)
        pltpu.sync_copy(kth_buf, kth_hbm.at[pl.ds(base, CHUNK)])


def _oem(lo, n, r):
    step = r * 2
    pairs = []
    if step < n:
        pairs += _oem(lo, n, step)
        pairs += _oem(lo + r, n, step)
        pairs += [(i, i + r) for i in range(lo + r, lo + n - r, step)]
    else:
        pairs.append((lo, lo + r))
    return pairs


def _oems(lo, n):
    pairs = []
    if n > 1:
        m = n // 2
        pairs += _oems(lo, m)
        pairs += _oems(lo + m, m)
        pairs += _oem(lo, n, 1)
    return pairs


def _prune(net, needed):
    needed = set(needed)
    out = []
    for i, j in reversed(net):
        if i in needed or j in needed:
            out.append((i, j))
            needed.add(i)
            needed.add(j)
    return list(reversed(out))


_SORT16 = _oems(0, 16)
_SORT16_LOW9 = _prune(_SORT16, range(K))
_SORT9 = [(i, j) for (i, j) in _SORT16 if i < K and j < K]


def _apply_net(net, v):
    v = list(v)
    for i, j in net:
        lo = jnp.minimum(v[i], v[j])
        hi = jnp.maximum(v[i], v[j])
        v[i], v[j] = lo, hi
    return v


TC_BC = 8192
TILE = 128


def _tc_body(x_ref, kth_ref, zero_ref):
    zero_ref[...] = jnp.zeros((8, 128), jnp.int32)
    for t in range(TC_BC // TILE):
        cs = pl.ds(t * TILE, TILE)
        v = [x_ref[pl.ds(rg * 8, 8), cs] for rg in range(16)]
        v = _apply_net(_SORT16_LOW9, v)
        c = v[:K]
        half = 4
        while half >= 1:
            a = [c[i][0:half] for i in range(K)]
            b = [c[i][half:2 * half] for i in range(K)]
            c = [jnp.minimum(a[i], b[K - 1 - i]) for i in range(K)]
            if half > 1:
                c = _apply_net(_SORT9, c)
            half //= 2
        kth = c[0]
        for i in range(1, K):
            kth = jnp.maximum(kth, c[i])
        kth_ref[0, 0:1, cs] = kth


def _tc_call(x, ncols):
    nblk = ncols // TC_BC
    return pl.pallas_call(
        _tc_body,
        grid=(nblk,),
        in_specs=[pl.BlockSpec((ROWS, TC_BC), lambda i: (0, i))],
        out_specs=[
            pl.BlockSpec((1, 8, TC_BC), lambda i: (i, 0, 0)),
            pl.BlockSpec((8, 128), lambda i: (0, 0)),
        ],
        out_shape=[
            jax.ShapeDtypeStruct((nblk, 8, TC_BC), jnp.float32),
            jax.ShapeDtypeStruct((8, 128), jnp.int32),
        ],
    )(x)


_SC_CALL_CACHE = {}


def _sc_call(x):
    if "call" not in _SC_CALL_CACHE:
        mesh = plsc.VectorSubcoreMesh(core_axis_name="c", subcore_axis_name="s")
        _SC_CALL_CACHE["call"] = pl.kernel(
            _sc_body,
            out_type=[
                jax.ShapeDtypeStruct((COLS,), jnp.float32),
                jax.ShapeDtypeStruct((LANES,), jnp.int32),
            ],
            mesh=mesh,
            scratch_types=[
                pltpu.VMEM((ROWS, CHUNK), jnp.float32),
                pltpu.VMEM((CHUNK,), jnp.float32),
                pltpu.VMEM((LANES,), jnp.int32),
            ],
        )
    return _SC_CALL_CACHE["call"](x)


def kernel(x):
    kth_tc, zero_tc = _tc_call(x, COLS)
    del kth_tc
    return zero_tc[0, 0]
